# baseline (speedup 1.0000x reference)
import jax
import jax.numpy as jnp
from jax import lax
from jax.experimental import pallas as pl
from jax.experimental.pallas import tpu as pltpu
from jax.experimental.pallas import tpu_sc as plsc

N_ENTS = 50000
N_RELS = 256
N_FACTS = 800000
B = 16
MAX_HOPS = 3

NC = 2
NS = 16
L = 16
BPC = B // NC
NHALF = 2
FPW = N_FACTS // NHALF
CH = 4000
NCHUNK = FPW // CH
ITERS = CH // L
NVEC = N_ENTS // L
SPLIT = (NVEC // 2) * L
RNG = (SPLIT, N_ENTS - SPLIT)
RVEC = (SPLIT // L, NVEC - SPLIT // L)


def _walk_body(ht_hbm, rel_hbm, rels_hbm, init_hbm, zeros_hbm,
               out_hbm, xchg_hbm, sums_hbm,
               e_v, w_v, relv, sv, sv2,
               hb0, hb1, rb0, rb1,
               sem0, sem1, semz):
    c = lax.axis_index("c")
    s = lax.axis_index("s")
    local_b = s % BPC
    batch = c * BPC + local_b
    half = s // BPC
    fbase = half * FPW

    slots = ((hb0, rb0, sem0), (hb1, rb1, sem1))

    def issue(j, slot):
        hb, rb, sem = slot
        off = fbase + j * CH
        pltpu.async_copy(ht_hbm.at[pl.ds(off, CH)], hb, sem)
        pltpu.async_copy(rel_hbm.at[pl.ds(off, CH)], rb, sem)

    def drain(slot):
        hb, rb, sem = slot
        pltpu.make_async_copy(ht_hbm.at[pl.ds(0, CH)], hb, sem).wait()
        pltpu.make_async_copy(rel_hbm.at[pl.ds(0, CH)], rb, sem).wait()

    pltpu.sync_copy(init_hbm.at[pl.ds(batch * N_ENTS, N_ENTS)], e_v)
    pltpu.sync_copy(zeros_hbm, w_v)

    zvec = jnp.zeros((L,), jnp.float32)
    iota = lax.iota(jnp.int32, L)

    for hop in range(MAX_HOPS):
        pltpu.sync_copy(
            rels_hbm.at[pl.ds((batch * MAX_HOPS + hop) * (N_RELS * L),
                              N_RELS * L)],
            relv)

        if hop > 0:
            pltpu.make_async_copy(zeros_hbm, w_v, semz).wait()

        issue(0, slots[0])
        issue(1, slots[1])

        def chunk_pass(jj, _):
            jo = jj * 2
            for bslot in range(2):
                slot = slots[bslot]
                hb, rb, _sem = slot
                drain(slot)

                @plsc.parallel_loop(0, ITERS, unroll=10)
                def _(i):
                    base = i * L
                    htv = hb[pl.ds(base, L)]
                    rv = rb[pl.ds(base, L)]
                    hv = lax.shift_right_logical(htv, 16)
                    tv = htv & 0xFFFF
                    ridx = lax.shift_left(rv, 4) | iota
                    rf = plsc.load_gather(relv, [ridx])
                    ef = plsc.load_gather(e_v, [hv])
                    plsc.addupdate_scatter(w_v, [tv], rf * ef)

                nxt = jo + bslot + 2

                @pl.when(nxt < NCHUNK)
                def _():
                    issue(nxt, slot)
            return _
        lax.fori_loop(0, NCHUNK // 2, chunk_pass, None)

        xbase = batch * N_ENTS
        obase = batch * (MAX_HOPS * N_ENTS) + hop * N_ENTS
        LO = (0, SPLIT)

        for h in range(2):
            olo, on = LO[1 - h], RNG[1 - h]

            @pl.when(half == h)
            def _(olo=olo, on=on):
                pltpu.sync_copy(w_v.at[pl.ds(olo, on)],
                                xchg_hbm.at[pl.ds(xbase + olo, on)])
        plsc.subcore_barrier()

        for h in range(2):
            lo, n, nv = LO[h], RNG[h], RVEC[h]

            @pl.when(half == h)
            def _(lo=lo, n=n, nv=nv, h=h):
                pltpu.sync_copy(xchg_hbm.at[pl.ds(xbase + lo, n)],
                                e_v.at[pl.ds(lo, n)])
                b0 = lo // L

                def comb_body(i, acc):
                    sl = pl.ds((b0 + i) * L, L)
                    v = e_v[sl] + w_v[sl]
                    e_v[sl] = v
                    return acc + v
                acc = lax.fori_loop(0, nv, comb_body, zvec, unroll=5)
                if hop < MAX_HOPS - 1:
                    pltpu.async_copy(zeros_hbm, w_v, semz)
                sv[...] = acc
                pltpu.sync_copy(
                    sv, sums_hbm.at[pl.ds((batch * NHALF + h) * L, L)])
        plsc.subcore_barrier()

        for h in range(2):
            lo, n, nv = LO[h], RNG[h], RVEC[h]

            @pl.when(half == h)
            def _(lo=lo, n=n, nv=nv, h=h):
                pltpu.sync_copy(
                    sums_hbm.at[pl.ds((batch * NHALF + (1 - h)) * L, L)], sv2)
                total = jnp.sum(sv[...] + sv2[...])
                inv = 1.0 / (lax.broadcast(total, (L,)) + 1e-6)
                b0 = lo // L

                def norm_body(i, _n2):
                    sl = pl.ds((b0 + i) * L, L)
                    e_v[sl] = e_v[sl] * inv
                    return _n2
                lax.fori_loop(0, nv, norm_body, None, unroll=5)
                pltpu.sync_copy(e_v.at[pl.ds(lo, n)],
                                out_hbm.at[pl.ds(obase + lo, n)])
                pltpu.sync_copy(e_v.at[pl.ds(lo, n)],
                                xchg_hbm.at[pl.ds(xbase + lo, n)])
        plsc.subcore_barrier()

        for h in range(2):
            olo, on = LO[1 - h], RNG[1 - h]

            @pl.when(half == h)
            def _(olo=olo, on=on):
                pltpu.sync_copy(xchg_hbm.at[pl.ds(xbase + olo, on)],
                                e_v.at[pl.ds(olo, on)])
        plsc.subcore_barrier()


def _make_walk():
    return pl.kernel(
        _walk_body,
        out_type=(
            jax.ShapeDtypeStruct((B * MAX_HOPS * N_ENTS,), jnp.float32),
            jax.ShapeDtypeStruct((B * N_ENTS,), jnp.float32),
            jax.ShapeDtypeStruct((B * NHALF * L,), jnp.float32),
        ),
        compiler_params=pltpu.CompilerParams(needs_layout_passes=False),
        mesh=plsc.VectorSubcoreMesh(
            core_axis_name="c", subcore_axis_name="s",
            num_cores=NC, num_subcores=NS),
        scratch_types=[
            pltpu.VMEM((N_ENTS,), jnp.float32),
            pltpu.VMEM((N_ENTS,), jnp.float32),
            pltpu.VMEM((N_RELS * L,), jnp.float32),
            pltpu.VMEM((L,), jnp.float32),
            pltpu.VMEM((L,), jnp.float32),
            pltpu.VMEM((CH,), jnp.int32),
            pltpu.VMEM((CH,), jnp.int32),
            pltpu.VMEM((CH,), jnp.int32),
            pltpu.VMEM((CH,), jnp.int32),
            pltpu.SemaphoreType.DMA,
            pltpu.SemaphoreType.DMA,
            pltpu.SemaphoreType.DMA,
        ],
    )


@jax.jit
def kernel(head_idx, rel_idx, tail_idx, rels_seq, init_ent):
    ht = lax.shift_left(head_idx, 16) | tail_idx
    rels_rep = jnp.broadcast_to(rels_seq[..., None], (B, MAX_HOPS, N_RELS, L))
    walked, _xchg, _sums = _make_walk()(
        ht, rel_idx, rels_rep.reshape(-1), init_ent.reshape(-1),
        jnp.zeros((N_ENTS,), jnp.float32))
    walked = walked.reshape(B, MAX_HOPS, N_ENTS)
    return jnp.concatenate([init_ent[:, None, :], walked], axis=1)


# speedup vs baseline: 1.0076x; 1.0076x over previous
import jax
import jax.numpy as jnp
from jax import lax
from jax.experimental import pallas as pl
from jax.experimental.pallas import tpu as pltpu
from jax.experimental.pallas import tpu_sc as plsc

N_ENTS = 50000
N_RELS = 256
N_FACTS = 800000
B = 16
MAX_HOPS = 3

NC = 2
NS = 16
L = 16
BPC = B // NC
NHALF = 2
FPW = N_FACTS // NHALF
CH = 4000
NCHUNK = FPW // CH
ITERS = CH // L
NVEC = N_ENTS // L
SPLIT = (NVEC // 2) * L
RNG = (SPLIT, N_ENTS - SPLIT)
RVEC = (SPLIT // L, NVEC - SPLIT // L)


def _walk_body(ht_hbm, rel_hbm, rels_hbm, init_hbm,
               out_hbm, xchg_hbm, sums_hbm,
               e_v, w_v, relv, sv, sv2,
               hb0, hb1, rb0, rb1,
               sem0, sem1):
    c = lax.axis_index("c")
    s = lax.axis_index("s")
    local_b = s % BPC
    batch = c * BPC + local_b
    half = s // BPC
    fbase = half * FPW

    slots = ((hb0, rb0, sem0), (hb1, rb1, sem1))

    def issue(j, slot):
        hb, rb, sem = slot
        off = fbase + j * CH
        pltpu.async_copy(ht_hbm.at[pl.ds(off, CH)], hb, sem)
        pltpu.async_copy(rel_hbm.at[pl.ds(off, CH)], rb, sem)

    def drain(slot):
        hb, rb, sem = slot
        pltpu.make_async_copy(ht_hbm.at[pl.ds(0, CH)], hb, sem).wait()
        pltpu.make_async_copy(rel_hbm.at[pl.ds(0, CH)], rb, sem).wait()

    pltpu.sync_copy(init_hbm.at[pl.ds(batch * N_ENTS, N_ENTS)], e_v)

    zvec = jnp.zeros((L,), jnp.float32)
    iota = lax.iota(jnp.int32, L)

    for hop in range(MAX_HOPS):
        pltpu.sync_copy(
            rels_hbm.at[pl.ds((batch * MAX_HOPS + hop) * (N_RELS * L),
                              N_RELS * L)],
            relv)

        def zero_body(i, _):
            w_v[pl.ds(i * L, L)] = zvec
            return _
        lax.fori_loop(0, NVEC, zero_body, None, unroll=5)

        issue(0, slots[0])
        issue(1, slots[1])

        def chunk_pass(jj, _):
            jo = jj * 2
            for bslot in range(2):
                slot = slots[bslot]
                hb, rb, _sem = slot
                drain(slot)

                @plsc.parallel_loop(0, ITERS, unroll=10)
                def _(i):
                    base = i * L
                    htv = hb[pl.ds(base, L)]
                    rv = rb[pl.ds(base, L)]
                    hv = lax.shift_right_logical(htv, 16)
                    tv = htv & 0xFFFF
                    ridx = lax.shift_left(rv, 4) | iota
                    rf = plsc.load_gather(relv, [ridx])
                    ef = plsc.load_gather(e_v, [hv])
                    plsc.addupdate_scatter(w_v, [tv], rf * ef)

                nxt = jo + bslot + 2

                @pl.when(nxt < NCHUNK)
                def _():
                    issue(nxt, slot)
            return _
        lax.fori_loop(0, NCHUNK // 2, chunk_pass, None)

        xbase = batch * N_ENTS
        obase = batch * (MAX_HOPS * N_ENTS) + hop * N_ENTS
        LO = (0, SPLIT)

        for h in range(2):
            olo, on = LO[1 - h], RNG[1 - h]

            @pl.when(half == h)
            def _(olo=olo, on=on):
                pltpu.sync_copy(w_v.at[pl.ds(olo, on)],
                                xchg_hbm.at[pl.ds(xbase + olo, on)])
        plsc.subcore_barrier()

        for h in range(2):
            lo, n, nv = LO[h], RNG[h], RVEC[h]

            @pl.when(half == h)
            def _(lo=lo, n=n, nv=nv, h=h):
                pltpu.sync_copy(xchg_hbm.at[pl.ds(xbase + lo, n)],
                                e_v.at[pl.ds(lo, n)])
                b0 = lo // L

                def comb_body(i, acc):
                    sl = pl.ds((b0 + i) * L, L)
                    v = e_v[sl] + w_v[sl]
                    e_v[sl] = v
                    return acc + v
                acc = lax.fori_loop(0, nv, comb_body, zvec, unroll=5)
                sv[...] = acc
                pltpu.sync_copy(
                    sv, sums_hbm.at[pl.ds((batch * NHALF + h) * L, L)])
        plsc.subcore_barrier()

        for h in range(2):
            lo, n, nv = LO[h], RNG[h], RVEC[h]

            @pl.when(half == h)
            def _(lo=lo, n=n, nv=nv, h=h):
                pltpu.sync_copy(
                    sums_hbm.at[pl.ds((batch * NHALF + (1 - h)) * L, L)], sv2)
                total = jnp.sum(sv[...] + sv2[...])
                inv = 1.0 / (lax.broadcast(total, (L,)) + 1e-6)
                b0 = lo // L

                def norm_body(i, _n2):
                    sl = pl.ds((b0 + i) * L, L)
                    e_v[sl] = e_v[sl] * inv
                    return _n2
                lax.fori_loop(0, nv, norm_body, None, unroll=5)
                pltpu.sync_copy(e_v.at[pl.ds(lo, n)],
                                out_hbm.at[pl.ds(obase + lo, n)])
                pltpu.sync_copy(e_v.at[pl.ds(lo, n)],
                                xchg_hbm.at[pl.ds(xbase + lo, n)])
        plsc.subcore_barrier()

        for h in range(2):
            olo, on = LO[1 - h], RNG[1 - h]

            @pl.when(half == h)
            def _(olo=olo, on=on):
                pltpu.sync_copy(xchg_hbm.at[pl.ds(xbase + olo, on)],
                                e_v.at[pl.ds(olo, on)])
        plsc.subcore_barrier()


def _make_walk():
    return pl.kernel(
        _walk_body,
        out_type=(
            jax.ShapeDtypeStruct((B * MAX_HOPS * N_ENTS,), jnp.float32),
            jax.ShapeDtypeStruct((B * N_ENTS,), jnp.float32),
            jax.ShapeDtypeStruct((B * NHALF * L,), jnp.float32),
        ),
        compiler_params=pltpu.CompilerParams(needs_layout_passes=False),
        mesh=plsc.VectorSubcoreMesh(
            core_axis_name="c", subcore_axis_name="s",
            num_cores=NC, num_subcores=NS),
        scratch_types=[
            pltpu.VMEM((N_ENTS,), jnp.float32),
            pltpu.VMEM((N_ENTS,), jnp.float32),
            pltpu.VMEM((N_RELS * L,), jnp.float32),
            pltpu.VMEM((L,), jnp.float32),
            pltpu.VMEM((L,), jnp.float32),
            pltpu.VMEM((CH,), jnp.int32),
            pltpu.VMEM((CH,), jnp.int32),
            pltpu.VMEM((CH,), jnp.int32),
            pltpu.VMEM((CH,), jnp.int32),
            pltpu.SemaphoreType.DMA,
            pltpu.SemaphoreType.DMA,
        ],
    )


@jax.jit
def kernel(head_idx, rel_idx, tail_idx, rels_seq, init_ent):
    ht = lax.shift_left(head_idx, 16) | tail_idx
    rels_rep = jnp.broadcast_to(rels_seq[..., None], (B, MAX_HOPS, N_RELS, L))
    walked, _xchg, _sums = _make_walk()(
        ht, rel_idx, rels_rep.reshape(-1), init_ent.reshape(-1))
    walked = walked.reshape(B, MAX_HOPS, N_ENTS)
    return jnp.concatenate([init_ent[:, None, :], walked], axis=1)
